# baseline (speedup 1.0000x reference)
import jax
import jax.numpy as jnp
from jax import lax
from jax.experimental import pallas as pl
from jax.experimental.pallas import tpu as pltpu
from jax.experimental.pallas import tpu_sc as plsc

_NS = 16
_CHUNK = 128


def _unpool(pooled_H, idx2, n_rows):
    P, F = pooled_H.shape
    rows_per_w = P // _NS
    zrows_per_w = n_rows // _NS
    zbuf_rows = min(zrows_per_w, 128)
    n_zcopies = zrows_per_w // zbuf_rows
    n_chunks = rows_per_w // _CHUNK

    mesh = plsc.VectorSubcoreMesh(
        core_axis_name="c", subcore_axis_name="s", num_cores=1)

    def body(pooled_hbm, idx_hbm, h_hbm, zbuf, idxbuf, rows, sem):
        w = lax.axis_index("s")

        def zstep(i, carry):
            r = i // (F // 16)
            c = (i % (F // 16)) * 16
            zbuf[r, pl.ds(c, 16)] = jnp.zeros((16,), pooled_hbm.dtype)
            return carry
        lax.fori_loop(0, zbuf_rows * (F // 16), zstep, 0)

        for t in range(n_zcopies):
            pltpu.sync_copy(
                zbuf, h_hbm.at[pl.ds(w * zrows_per_w + t * zbuf_rows,
                                     zbuf_rows)])
        plsc.subcore_barrier()

        pltpu.sync_copy(idx_hbm.at[pl.ds(w * n_chunks, n_chunks)], idxbuf)
        pltpu.sync_copy(pooled_hbm.at[pl.ds(w * rows_per_w, rows_per_w)],
                        rows)
        copies = [
            pltpu.async_copy(rows.at[pl.ds(j * _CHUNK, _CHUNK)],
                             h_hbm.at[idxbuf.at[j]], sem)
            for j in range(n_chunks)
        ]
        for c in copies:
            c.wait()

    return pl.kernel(
        body,
        out_type=jax.ShapeDtypeStruct((n_rows, F), pooled_H.dtype),
        mesh=mesh,
        scratch_types=[
            pltpu.VMEM((zbuf_rows, F), pooled_H.dtype),
            pltpu.VMEM((n_chunks, _CHUNK), jnp.int32),
            pltpu.VMEM((rows_per_w, F), pooled_H.dtype),
            pltpu.SemaphoreType.DMA,
        ],
    )(pooled_H, idx2)


_COPY_BLOCK_ROWS = 128
_COPY_NBUF = 12
_COPY_LOOK = 6


def _tc_copy(a):
    n_rows, n_cols = a.shape
    n_chunks = n_rows // _COPY_BLOCK_ROWS
    nbuf = _COPY_NBUF
    look = _COPY_LOOK

    def body(a_hbm, out_hbm, buf, in_sem, out_sem):
        def chunk(ref, i):
            return ref.at[pl.ds(i * _COPY_BLOCK_ROWS, _COPY_BLOCK_ROWS)]

        in_d = [None] * n_chunks
        out_d = [None] * n_chunks
        for j in range(look):
            in_d[j] = pltpu.async_copy(chunk(a_hbm, j), buf.at[j % nbuf],
                                       in_sem.at[j % nbuf])
        waited = set()
        for i in range(n_chunks):
            j = i + look
            if j < n_chunks:
                p = j - nbuf
                if p >= 0:
                    out_d[p].wait()
                    waited.add(p)
                in_d[j] = pltpu.async_copy(chunk(a_hbm, j),
                                           buf.at[j % nbuf],
                                           in_sem.at[j % nbuf])
            in_d[i].wait()
            out_d[i] = pltpu.async_copy(buf.at[i % nbuf], chunk(out_hbm, i),
                                        out_sem.at[i % nbuf])
        for i in range(n_chunks):
            if i not in waited:
                out_d[i].wait()

    return pl.pallas_call(
        body,
        out_shape=jax.ShapeDtypeStruct((n_rows, n_cols), a.dtype),
        in_specs=[pl.BlockSpec(memory_space=pl.ANY)],
        out_specs=pl.BlockSpec(memory_space=pl.ANY),
        scratch_shapes=[
            pltpu.VMEM((nbuf, _COPY_BLOCK_ROWS, n_cols), a.dtype),
            pltpu.SemaphoreType.DMA((nbuf,)),
            pltpu.SemaphoreType.DMA((nbuf,)),
        ],
    )(a)


def kernel(pooled_H, A_old, idx):
    P, F = pooled_H.shape
    n_rows = A_old.shape[0]
    idx2 = idx.reshape(P // _CHUNK, _CHUNK)
    H = _unpool(pooled_H, idx2, n_rows)
    return (H, _tc_copy(A_old))


# speedup vs baseline: 1.0019x; 1.0019x over previous
import jax
import jax.numpy as jnp
from jax import lax
from jax.experimental import pallas as pl
from jax.experimental.pallas import tpu as pltpu
from jax.experimental.pallas import tpu_sc as plsc

_NS = 16
_CHUNK = 128


def _unpool(pooled_H, idx2, n_rows):
    P, F = pooled_H.shape
    rows_per_w = P // _NS
    zrows_per_w = n_rows // _NS
    zbuf_rows = min(zrows_per_w, 128)
    n_zcopies = zrows_per_w // zbuf_rows
    n_chunks = rows_per_w // _CHUNK

    mesh = plsc.VectorSubcoreMesh(
        core_axis_name="c", subcore_axis_name="s", num_cores=1)

    def body(pooled_hbm, idx_hbm, h_hbm, zbuf, idxbuf, rows, sem):
        w = lax.axis_index("s")

        def zstep(i, carry):
            r = i // (F // 16)
            c = (i % (F // 16)) * 16
            zbuf[r, pl.ds(c, 16)] = jnp.zeros((16,), pooled_hbm.dtype)
            return carry
        lax.fori_loop(0, zbuf_rows * (F // 16), zstep, 0)

        for t in range(n_zcopies):
            pltpu.sync_copy(
                zbuf, h_hbm.at[pl.ds(w * zrows_per_w + t * zbuf_rows,
                                     zbuf_rows)])
        plsc.subcore_barrier()

        pltpu.sync_copy(idx_hbm.at[pl.ds(w * n_chunks, n_chunks)], idxbuf)
        pltpu.sync_copy(pooled_hbm.at[pl.ds(w * rows_per_w, rows_per_w)],
                        rows)
        copies = [
            pltpu.async_copy(rows.at[pl.ds(j * _CHUNK, _CHUNK)],
                             h_hbm.at[idxbuf.at[j]], sem)
            for j in range(n_chunks)
        ]
        for c in copies:
            c.wait()

    return pl.kernel(
        body,
        out_type=jax.ShapeDtypeStruct((n_rows, F), pooled_H.dtype),
        mesh=mesh,
        scratch_types=[
            pltpu.VMEM((zbuf_rows, F), pooled_H.dtype),
            pltpu.VMEM((n_chunks, _CHUNK), jnp.int32),
            pltpu.VMEM((rows_per_w, F), pooled_H.dtype),
            pltpu.SemaphoreType.DMA,
        ],
    )(pooled_H, idx2)


_COPY_BLOCK_ROWS = 512
_COPY_NBUF = 3
_COPY_LOOK = 2


def _tc_copy(a):
    n_rows, n_cols = a.shape
    n_chunks = n_rows // _COPY_BLOCK_ROWS
    nbuf = _COPY_NBUF
    look = _COPY_LOOK

    def body(a_hbm, out_hbm, buf, in_sem, out_sem):
        def chunk(ref, i):
            return ref.at[pl.ds(i * _COPY_BLOCK_ROWS, _COPY_BLOCK_ROWS)]

        in_d = [None] * n_chunks
        out_d = [None] * n_chunks
        for j in range(look):
            in_d[j] = pltpu.async_copy(chunk(a_hbm, j), buf.at[j % nbuf],
                                       in_sem.at[j % nbuf])
        waited = set()
        for i in range(n_chunks):
            j = i + look
            if j < n_chunks:
                p = j - nbuf
                if p >= 0:
                    out_d[p].wait()
                    waited.add(p)
                in_d[j] = pltpu.async_copy(chunk(a_hbm, j),
                                           buf.at[j % nbuf],
                                           in_sem.at[j % nbuf])
            in_d[i].wait()
            out_d[i] = pltpu.async_copy(buf.at[i % nbuf], chunk(out_hbm, i),
                                        out_sem.at[i % nbuf])
        for i in range(n_chunks):
            if i not in waited:
                out_d[i].wait()

    return pl.pallas_call(
        body,
        out_shape=jax.ShapeDtypeStruct((n_rows, n_cols), a.dtype),
        in_specs=[pl.BlockSpec(memory_space=pl.ANY)],
        out_specs=pl.BlockSpec(memory_space=pl.ANY),
        scratch_shapes=[
            pltpu.VMEM((nbuf, _COPY_BLOCK_ROWS, n_cols), a.dtype),
            pltpu.SemaphoreType.DMA((nbuf,)),
            pltpu.SemaphoreType.DMA((nbuf,)),
        ],
    )(a)


def kernel(pooled_H, A_old, idx):
    P, F = pooled_H.shape
    n_rows = A_old.shape[0]
    idx2 = idx.reshape(P // _CHUNK, _CHUNK)
    H = _unpool(pooled_H, idx2, n_rows)
    return (H, _tc_copy(A_old))


# speedup vs baseline: 1.0021x; 1.0003x over previous
import jax
import jax.numpy as jnp
from jax import lax
from jax.experimental import pallas as pl
from jax.experimental.pallas import tpu as pltpu
from jax.experimental.pallas import tpu_sc as plsc

_NS = 16
_CHUNK = 128


def _unpool(pooled_H, idx2, n_rows):
    P, F = pooled_H.shape
    rows_per_w = P // _NS
    zrows_per_w = n_rows // _NS
    zbuf_rows = min(zrows_per_w, 128)
    n_zcopies = zrows_per_w // zbuf_rows
    n_chunks = rows_per_w // _CHUNK

    mesh = plsc.VectorSubcoreMesh(
        core_axis_name="c", subcore_axis_name="s", num_cores=1)

    def body(pooled_hbm, idx_hbm, h_hbm, zbuf, idxbuf, rows, sem):
        w = lax.axis_index("s")

        def zstep(i, carry):
            r = i // (F // 16)
            c = (i % (F // 16)) * 16
            zbuf[r, pl.ds(c, 16)] = jnp.zeros((16,), pooled_hbm.dtype)
            return carry
        lax.fori_loop(0, zbuf_rows * (F // 16), zstep, 0)

        base = w * zrows_per_w
        for t in range(n_zcopies):
            pltpu.sync_copy(
                zbuf, h_hbm.at[pl.ds(base + t * zbuf_rows, zbuf_rows)])
        plsc.subcore_barrier()

        pltpu.sync_copy(idx_hbm.at[pl.ds(w * n_chunks, n_chunks)], idxbuf)
        pltpu.sync_copy(pooled_hbm.at[pl.ds(w * rows_per_w, rows_per_w)],
                        rows)
        copies = [
            pltpu.async_copy(rows.at[pl.ds(j * _CHUNK, _CHUNK)],
                             h_hbm.at[idxbuf.at[j]], sem)
            for j in range(n_chunks)
        ]
        for c in copies:
            c.wait()

    return pl.kernel(
        body,
        out_type=jax.ShapeDtypeStruct((n_rows, F), pooled_H.dtype),
        mesh=mesh,
        scratch_types=[
            pltpu.VMEM((zbuf_rows, F), pooled_H.dtype),
            pltpu.VMEM((n_chunks, _CHUNK), jnp.int32),
            pltpu.VMEM((rows_per_w, F), pooled_H.dtype),
            pltpu.SemaphoreType.DMA,
        ],
    )(pooled_H, idx2)


_COPY_BLOCK_ROWS = 512
_COPY_NBUF = 3
_COPY_LOOK = 2


def _tc_copy(a):
    n_rows, n_cols = a.shape
    n_chunks = n_rows // _COPY_BLOCK_ROWS
    nbuf = _COPY_NBUF
    look = _COPY_LOOK

    def body(a_hbm, out_hbm, buf, in_sem, out_sem):
        def chunk(ref, i):
            return ref.at[pl.ds(i * _COPY_BLOCK_ROWS, _COPY_BLOCK_ROWS)]

        in_d = [None] * n_chunks
        out_d = [None] * n_chunks
        for j in range(look):
            in_d[j] = pltpu.async_copy(chunk(a_hbm, j), buf.at[j % nbuf],
                                       in_sem.at[j % nbuf])
        waited = set()
        for i in range(n_chunks):
            j = i + look
            if j < n_chunks:
                p = j - nbuf
                if p >= 0:
                    out_d[p].wait()
                    waited.add(p)
                in_d[j] = pltpu.async_copy(chunk(a_hbm, j),
                                           buf.at[j % nbuf],
                                           in_sem.at[j % nbuf])
            in_d[i].wait()
            out_d[i] = pltpu.async_copy(buf.at[i % nbuf], chunk(out_hbm, i),
                                        out_sem.at[i % nbuf])
        for i in range(n_chunks):
            if i not in waited:
                out_d[i].wait()

    return pl.pallas_call(
        body,
        out_shape=jax.ShapeDtypeStruct((n_rows, n_cols), a.dtype),
        in_specs=[pl.BlockSpec(memory_space=pl.ANY)],
        out_specs=pl.BlockSpec(memory_space=pl.ANY),
        scratch_shapes=[
            pltpu.VMEM((nbuf, _COPY_BLOCK_ROWS, n_cols), a.dtype),
            pltpu.SemaphoreType.DMA((nbuf,)),
            pltpu.SemaphoreType.DMA((nbuf,)),
        ],
    )(a)


def kernel(pooled_H, A_old, idx):
    P, F = pooled_H.shape
    n_rows = A_old.shape[0]
    idx2 = idx.reshape(P // _CHUNK, _CHUNK)
    H = _unpool(pooled_H, idx2, n_rows)
    return (H, _tc_copy(A_old))
